# baseline (speedup 1.0000x reference)
import jax
import jax.numpy as jnp
from jax.experimental import pallas as pl
from jax.experimental.pallas import tpu as pltpu

_N = 8192
_D = 2048
_ALPHA = 1.0
_BR = 256


def _tc_body(t_ref, m_ref, d_ref, o_ref):
    d = d_ref[...]
    t = t_ref[...]
    m = m_ref[...]

    d1 = d[0:1, :]
    d2 = d[1:2, :]
    d3 = d[2:3, :]
    n1 = jnp.sum(d1 * d1)
    n2 = jnp.sum(d2 * d2)
    n3 = jnp.sum(d3 * d3)
    g12 = jnp.sum(d1 * d2)
    g13 = jnp.sum(d1 * d3)
    g23 = jnp.sum(d2 * d3)

    c1 = jnp.sum(t * d1, axis=1, keepdims=True)
    c2 = jnp.sum(t * d2, axis=1, keepdims=True)
    c3 = jnp.sum(t * d3, axis=1, keepdims=True)

    a1 = c1 / n1
    a2 = (c2 - a1 * g12) / n2
    a3 = (c3 - a1 * g13 - a2 * g23) / n3

    comb = a1 * d1 + a2 * d2 + (a3 - _ALPHA) * d3
    o_ref[...] = t - m * comb


def kernel(t_embs, token_mask, delta_front, delta_side, delta_back):
    m = token_mask.astype(jnp.float32).reshape(_N, 1)
    d = jnp.stack([delta_front, delta_side, delta_back], axis=0)
    grid = (_N // _BR,)
    return pl.pallas_call(
        _tc_body,
        grid=grid,
        in_specs=[
            pl.BlockSpec((_BR, _D), lambda i: (i, 0)),
            pl.BlockSpec((_BR, 1), lambda i: (i, 0)),
            pl.BlockSpec((3, _D), lambda i: (0, 0)),
        ],
        out_specs=pl.BlockSpec((_BR, _D), lambda i: (i, 0)),
        out_shape=jax.ShapeDtypeStruct((_N, _D), jnp.float32),
    )(t_embs, m, d)


# speedup vs baseline: 1.2005x; 1.2005x over previous
import jax
import jax.numpy as jnp
from jax.experimental import pallas as pl
from jax.experimental.pallas import tpu as pltpu

_N = 8192
_D = 2048
_ALPHA = 1.0
_BR = 1024


def _tc_body(t_ref, m_ref, d_ref, o_ref):
    d = d_ref[...]
    t = t_ref[...]
    m = m_ref[...]

    d1 = d[0:1, :]
    d2 = d[1:2, :]
    d3 = d[2:3, :]
    n1 = jnp.sum(d1 * d1)
    n2 = jnp.sum(d2 * d2)
    n3 = jnp.sum(d3 * d3)
    g12 = jnp.sum(d1 * d2)
    g13 = jnp.sum(d1 * d3)
    g23 = jnp.sum(d2 * d3)

    c1 = jnp.sum(t * d1, axis=1, keepdims=True)
    c2 = jnp.sum(t * d2, axis=1, keepdims=True)
    c3 = jnp.sum(t * d3, axis=1, keepdims=True)

    a1 = c1 / n1
    a2 = (c2 - a1 * g12) / n2
    a3 = (c3 - a1 * g13 - a2 * g23) / n3

    comb = a1 * d1 + a2 * d2 + (a3 - _ALPHA) * d3
    o_ref[...] = t - m * comb


def kernel(t_embs, token_mask, delta_front, delta_side, delta_back):
    m = token_mask.astype(jnp.float32).reshape(_N, 1)
    d = jnp.stack([delta_front, delta_side, delta_back], axis=0)
    grid = (_N // _BR,)
    return pl.pallas_call(
        _tc_body,
        grid=grid,
        in_specs=[
            pl.BlockSpec((_BR, _D), lambda i: (i, 0)),
            pl.BlockSpec((_BR, 1), lambda i: (i, 0)),
            pl.BlockSpec((3, _D), lambda i: (0, 0)),
        ],
        out_specs=pl.BlockSpec((_BR, _D), lambda i: (i, 0)),
        out_shape=jax.ShapeDtypeStruct((_N, _D), jnp.float32),
    )(t_embs, m, d)
